# baseline (speedup 1.0000x reference)
import dataclasses
import functools

import jax
import jax.numpy as jnp
from jax import lax
from jax.experimental import pallas as pl
from jax.experimental.pallas import tpu as pltpu
from jax.experimental.pallas import tpu_sc as plsc

H = W = 512
TH = TW = 64
GRID = 8
BINS = 256
CLIP_LIMIT = 4.0
CONTRAST_FACTOR = 1.15
BLEND_FACTOR = 0.65


def _per_tile_reduce(x, red, fill):
    lane = lax.broadcasted_iota(jnp.int32, x.shape, x.ndim - 1)
    row = lax.broadcasted_iota(jnp.int32, x.shape, x.ndim - 2)
    g = (row // TH) * (W // TW) + lane // TW
    out = jnp.zeros_like(x)
    for t in range((x.shape[-2] // TH) * (W // TW)):
        m = g == t
        out = jnp.where(m, red(jnp.where(m, x, fill)), out)
    return out


def _k_prep(img_ref, idx_ref):
    img = img_ref[...]
    y = 0.299 * img[0] + 0.587 * img[1] + 0.114 * img[2]
    i1 = jnp.clip(jnp.floor(y * BINS).astype(jnp.int32), 0, BINS - 1)
    tmin = _per_tile_reduce(y, jnp.min, jnp.float32(jnp.inf))
    tmax = _per_tile_reduce(y, jnp.max, jnp.float32(-jnp.inf))
    denom = jnp.where(tmax > tmin, tmax - tmin, 1.0)
    normalized = (y - tmin) / denom
    i2 = jnp.clip((normalized * 255).astype(jnp.int32), 0, 255)
    idx_ref[...] = i1 | (i2 << 8)


def _k_final(img_ref, eq_ref, out_ref):
    img = img_ref[...]
    y = 0.299 * img[0] + 0.587 * img[1] + 0.114 * img[2]
    eq = eq_ref[...]
    tmin = _per_tile_reduce(y, jnp.min, jnp.float32(jnp.inf))
    tmax = _per_tile_reduce(y, jnp.max, jnp.float32(-jnp.inf))
    has_range = tmax > tmin
    mean_val = _per_tile_reduce(eq, jnp.sum, jnp.float32(0.0)) * (
        1.0 / (TH * TW)
    )
    eq2 = (eq - mean_val) * CONTRAST_FACTOR + mean_val
    eq2 = jnp.clip(eq2, 0.0, 1.0)
    blended = y + BLEND_FACTOR * (eq2 - y)
    y_new = jnp.where(has_range, blended, y)
    safe = jnp.where(y > 0.01, y, 1.0)
    ratio = jnp.where(y > 0.01, y_new / safe, 1.0)
    out_ref[...] = jnp.clip(img * ratio[None, :, :], 0.0, 1.0)


def _sc_equalize(idx_packed):
    mesh = plsc.VectorSubcoreMesh(core_axis_name="c", subcore_axis_name="s")
    cp = pltpu.CompilerParams()
    if "needs_layout_passes" in pltpu.CompilerParams.__dataclass_fields__:
        cp = dataclasses.replace(cp, needs_layout_passes=False)

    @functools.partial(
        pl.kernel,
        out_type=jax.ShapeDtypeStruct((H, W), jnp.float32),
        mesh=mesh,
        compiler_params=cp,
        scratch_types=[
            pltpu.VMEM((TH, 2 * TW), jnp.int32),
            pltpu.VMEM((TH, 2 * TW), jnp.float32),
            pltpu.VMEM((2 * BINS,), jnp.float32),
            pltpu.VMEM((2 * BINS,), jnp.float32),
            pltpu.SemaphoreType.DMA,
        ],
    )
    def sck(idx_hbm, eq_hbm, idx_v, eq_v, hist_v, cdf_v, sem):
        wid = lax.axis_index("s") * 2 + lax.axis_index("c")
        r0 = (wid // 4) * TH
        c0 = (wid % 4) * (2 * TW)
        zeros16 = jnp.zeros((16,), jnp.float32)
        ones16 = jnp.ones((16,), jnp.float32)
        pltpu.async_copy(
            idx_hbm.at[pl.ds(r0, TH), pl.ds(c0, 2 * TW)], idx_v, sem
        ).wait()
        for i in range(2 * BINS // 16):
            hist_v[pl.ds(i * 16, 16)] = zeros16

        @pl.loop(0, TH, step=8)
        def _(r):
            for r8 in range(8):
                for cc in range(2 * TW // 16):
                    v = idx_v[r + r8, pl.ds(cc * 16, 16)]
                    b = jnp.bitwise_and(v, BINS - 1) + (cc // 4) * BINS
                    plsc.addupdate_scatter(hist_v, [b], ones16)

        for k in range(2):
            base = k * BINS
            excess = jnp.float32(0.0)
            for i in range(BINS // 16):
                h = hist_v[pl.ds(base + i * 16, 16)]
                excess = excess + jnp.sum(jnp.maximum(h - CLIP_LIMIT, 0.0))
            redist = excess * (1.0 / BINS)
            off = jnp.float32(0.0)
            for i in range(BINS // 16):
                h = hist_v[pl.ds(base + i * 16, 16)]
                h2 = jnp.minimum(h, CLIP_LIMIT) + redist
                c = plsc.cumsum(h2) + off
                cdf_v[pl.ds(base + i * 16, 16)] = c
                off = jnp.max(c)
            total = off
            for i in range(BINS // 16):
                cdf_v[pl.ds(base + i * 16, 16)] = (
                    cdf_v[pl.ds(base + i * 16, 16)] / total
                )

        @pl.loop(0, TH, step=8)
        def _(r):
            for r8 in range(8):
                for cc in range(2 * TW // 16):
                    v = idx_v[r + r8, pl.ds(cc * 16, 16)]
                    i2 = jnp.right_shift(v, 8) + (cc // 4) * BINS
                    eq_v[r + r8, pl.ds(cc * 16, 16)] = plsc.load_gather(
                        cdf_v, [i2]
                    )

        pltpu.async_copy(
            eq_v, eq_hbm.at[pl.ds(r0, TH), pl.ds(c0, 2 * TW)], sem
        ).wait()

    return sck(idx_packed)


@jax.jit
def kernel(img):
    idx = pl.pallas_call(
        _k_prep,
        grid=(GRID // 2,),
        in_specs=[pl.BlockSpec((3, 2 * TH, W), lambda i: (0, i, 0))],
        out_specs=pl.BlockSpec((2 * TH, W), lambda i: (i, 0)),
        out_shape=jax.ShapeDtypeStruct((H, W), jnp.int32),
    )(img)
    eq = _sc_equalize(idx)
    out = pl.pallas_call(
        _k_final,
        grid=(GRID // 2,),
        in_specs=[
            pl.BlockSpec((3, 2 * TH, W), lambda i: (0, i, 0)),
            pl.BlockSpec((2 * TH, W), lambda i: (i, 0)),
        ],
        out_specs=pl.BlockSpec((3, 2 * TH, W), lambda i: (0, i, 0)),
        out_shape=jax.ShapeDtypeStruct((3, H, W), jnp.float32),
    )(img, eq)
    return out


# speedup vs baseline: 1.2060x; 1.2060x over previous
import dataclasses
import functools

import jax
import jax.numpy as jnp
from jax import lax
from jax.experimental import pallas as pl
from jax.experimental.pallas import tpu as pltpu
from jax.experimental.pallas import tpu_sc as plsc

H = W = 512
TH = TW = 64
GRID = 8
BINS = 256
CLIP_LIMIT = 4.0
CONTRAST_FACTOR = 1.15
BLEND_FACTOR = 0.65


def _per_tile_reduce(x, red, fill):
    r0 = red(x, axis=0, keepdims=True)
    lane = lax.broadcasted_iota(jnp.int32, r0.shape, 1)
    g = lane // TW
    out = jnp.zeros_like(r0)
    for t in range(W // TW):
        m = g == t
        out = jnp.where(m, red(jnp.where(m, r0, fill)), out)
    return out


def _k_prep(img_ref, idx_ref):
    img = img_ref[...]
    y = 0.299 * img[0] + 0.587 * img[1] + 0.114 * img[2]
    i1 = jnp.clip(jnp.floor(y * BINS).astype(jnp.int32), 0, BINS - 1)
    tmin = _per_tile_reduce(y, jnp.min, jnp.float32(jnp.inf))
    tmax = _per_tile_reduce(y, jnp.max, jnp.float32(-jnp.inf))
    denom = jnp.where(tmax > tmin, tmax - tmin, 1.0)
    normalized = (y - tmin) / denom
    i2 = jnp.clip((normalized * 255).astype(jnp.int32), 0, 255)
    idx_ref[...] = i1 | (i2 << 8)


def _k_final(img_ref, eq_ref, out_ref):
    img = img_ref[...]
    y = 0.299 * img[0] + 0.587 * img[1] + 0.114 * img[2]
    eq = eq_ref[...]
    tmin = _per_tile_reduce(y, jnp.min, jnp.float32(jnp.inf))
    tmax = _per_tile_reduce(y, jnp.max, jnp.float32(-jnp.inf))
    has_range = tmax > tmin
    mean_val = _per_tile_reduce(eq, jnp.sum, jnp.float32(0.0)) * (
        1.0 / (TH * TW)
    )
    eq2 = (eq - mean_val) * CONTRAST_FACTOR + mean_val
    eq2 = jnp.clip(eq2, 0.0, 1.0)
    blended = y + BLEND_FACTOR * (eq2 - y)
    y_new = jnp.where(has_range, blended, y)
    safe = jnp.where(y > 0.01, y, 1.0)
    ratio = jnp.where(y > 0.01, y_new / safe, 1.0)
    out_ref[...] = jnp.clip(img * ratio[None, :, :], 0.0, 1.0)


def _sc_equalize(idx_packed):
    mesh = plsc.VectorSubcoreMesh(core_axis_name="c", subcore_axis_name="s")
    cp = pltpu.CompilerParams()
    if "needs_layout_passes" in pltpu.CompilerParams.__dataclass_fields__:
        cp = dataclasses.replace(cp, needs_layout_passes=False)

    @functools.partial(
        pl.kernel,
        out_type=jax.ShapeDtypeStruct((H, W), jnp.float32),
        mesh=mesh,
        compiler_params=cp,
        scratch_types=[
            pltpu.VMEM((TH, 2 * TW), jnp.int32),
            pltpu.VMEM((TH, 2 * TW), jnp.float32),
            pltpu.VMEM((2 * BINS,), jnp.float32),
            pltpu.VMEM((2 * BINS,), jnp.float32),
            pltpu.SemaphoreType.DMA,
        ],
    )
    def sck(idx_hbm, eq_hbm, idx_v, eq_v, hist_v, cdf_v, sem):
        wid = lax.axis_index("s") * 2 + lax.axis_index("c")
        r0 = (wid // 4) * TH
        c0 = (wid % 4) * (2 * TW)
        zeros16 = jnp.zeros((16,), jnp.float32)
        ones16 = jnp.ones((16,), jnp.float32)
        pltpu.async_copy(
            idx_hbm.at[pl.ds(r0, TH), pl.ds(c0, 2 * TW)], idx_v, sem
        ).wait()
        for i in range(2 * BINS // 16):
            hist_v[pl.ds(i * 16, 16)] = zeros16

        @pl.loop(0, TH, step=8)
        def _(r):
            for r8 in range(8):
                for cc in range(2 * TW // 16):
                    v = idx_v[r + r8, pl.ds(cc * 16, 16)]
                    b = jnp.bitwise_and(v, BINS - 1) + (cc // 4) * BINS
                    plsc.addupdate_scatter(hist_v, [b], ones16)

        for k in range(2):
            base = k * BINS
            excess = jnp.float32(0.0)
            for i in range(BINS // 16):
                h = hist_v[pl.ds(base + i * 16, 16)]
                excess = excess + jnp.sum(jnp.maximum(h - CLIP_LIMIT, 0.0))
            redist = excess * (1.0 / BINS)
            off = jnp.float32(0.0)
            for i in range(BINS // 16):
                h = hist_v[pl.ds(base + i * 16, 16)]
                h2 = jnp.minimum(h, CLIP_LIMIT) + redist
                c = plsc.cumsum(h2) + off
                cdf_v[pl.ds(base + i * 16, 16)] = c
                off = jnp.max(c)
            total = off
            for i in range(BINS // 16):
                cdf_v[pl.ds(base + i * 16, 16)] = (
                    cdf_v[pl.ds(base + i * 16, 16)] / total
                )

        @pl.loop(0, TH, step=8)
        def _(r):
            for r8 in range(8):
                for cc in range(2 * TW // 16):
                    v = idx_v[r + r8, pl.ds(cc * 16, 16)]
                    i2 = jnp.right_shift(v, 8) + (cc // 4) * BINS
                    eq_v[r + r8, pl.ds(cc * 16, 16)] = plsc.load_gather(
                        cdf_v, [i2]
                    )

        pltpu.async_copy(
            eq_v, eq_hbm.at[pl.ds(r0, TH), pl.ds(c0, 2 * TW)], sem
        ).wait()

    return sck(idx_packed)


@jax.jit
def kernel(img):
    idx = pl.pallas_call(
        _k_prep,
        grid=(GRID,),
        in_specs=[pl.BlockSpec((3, TH, W), lambda i: (0, i, 0))],
        out_specs=pl.BlockSpec((TH, W), lambda i: (i, 0)),
        out_shape=jax.ShapeDtypeStruct((H, W), jnp.int32),
    )(img)
    eq = _sc_equalize(idx)
    out = pl.pallas_call(
        _k_final,
        grid=(GRID,),
        in_specs=[
            pl.BlockSpec((3, TH, W), lambda i: (0, i, 0)),
            pl.BlockSpec((TH, W), lambda i: (i, 0)),
        ],
        out_specs=pl.BlockSpec((3, TH, W), lambda i: (0, i, 0)),
        out_shape=jax.ShapeDtypeStruct((3, H, W), jnp.float32),
    )(img, eq)
    return out


# speedup vs baseline: 1.2132x; 1.0060x over previous
import dataclasses
import functools

import jax
import jax.numpy as jnp
from jax import lax
from jax.experimental import pallas as pl
from jax.experimental.pallas import tpu as pltpu
from jax.experimental.pallas import tpu_sc as plsc

H = W = 512
TH = TW = 64
GRID = 8
BINS = 256
CLIP_LIMIT = 4.0
CONTRAST_FACTOR = 1.15
BLEND_FACTOR = 0.65


def _per_tile_reduce(x, red, fill):
    r0 = red(x, axis=0, keepdims=True)
    lane = lax.broadcasted_iota(jnp.int32, r0.shape, 1)
    g = lane // TW
    out = jnp.zeros_like(r0)
    for t in range(W // TW):
        m = g == t
        out = jnp.where(m, red(jnp.where(m, r0, fill)), out)
    return out


def _k_prep(img_ref, idx_ref):
    img = img_ref[...]
    y = 0.299 * img[0] + 0.587 * img[1] + 0.114 * img[2]
    i1 = jnp.clip(jnp.floor(y * BINS).astype(jnp.int32), 0, BINS - 1)
    tmin = _per_tile_reduce(y, jnp.min, jnp.float32(jnp.inf))
    tmax = _per_tile_reduce(y, jnp.max, jnp.float32(-jnp.inf))
    denom = jnp.where(tmax > tmin, tmax - tmin, 1.0)
    normalized = (y - tmin) / denom
    i2 = jnp.clip((normalized * 255).astype(jnp.int32), 0, 255)
    lane = lax.broadcasted_iota(jnp.int32, y.shape, y.ndim - 1)
    toff = (lane // TW) % 2 * BINS
    idx_ref[...] = (i1 + toff) | ((i2 + toff) << 9)


def _k_final(img_ref, eq_ref, out_ref):
    img = img_ref[...]
    y = 0.299 * img[0] + 0.587 * img[1] + 0.114 * img[2]
    eq = eq_ref[...]
    tmin = _per_tile_reduce(y, jnp.min, jnp.float32(jnp.inf))
    tmax = _per_tile_reduce(y, jnp.max, jnp.float32(-jnp.inf))
    has_range = tmax > tmin
    mean_val = _per_tile_reduce(eq, jnp.sum, jnp.float32(0.0)) * (
        1.0 / (TH * TW)
    )
    eq2 = (eq - mean_val) * CONTRAST_FACTOR + mean_val
    eq2 = jnp.clip(eq2, 0.0, 1.0)
    blended = y + BLEND_FACTOR * (eq2 - y)
    y_new = jnp.where(has_range, blended, y)
    safe = jnp.where(y > 0.01, y, 1.0)
    ratio = jnp.where(y > 0.01, y_new / safe, 1.0)
    out_ref[...] = jnp.clip(img * ratio[None, :, :], 0.0, 1.0)


def _sc_equalize(idx_packed):
    mesh = plsc.VectorSubcoreMesh(core_axis_name="c", subcore_axis_name="s")
    cp = pltpu.CompilerParams()
    if "needs_layout_passes" in pltpu.CompilerParams.__dataclass_fields__:
        cp = dataclasses.replace(cp, needs_layout_passes=False)

    @functools.partial(
        pl.kernel,
        out_type=jax.ShapeDtypeStruct((H, W), jnp.float32),
        mesh=mesh,
        compiler_params=cp,
        scratch_types=[
            pltpu.VMEM((TH, 2 * TW), jnp.int32),
            pltpu.VMEM((TH, 2 * TW), jnp.float32),
            pltpu.VMEM((2 * BINS,), jnp.float32),
            pltpu.VMEM((2 * BINS,), jnp.float32),
            pltpu.SemaphoreType.DMA,
        ],
    )
    def sck(idx_hbm, eq_hbm, idx_v, eq_v, hist_v, cdf_v, sem):
        wid = lax.axis_index("s") * 2 + lax.axis_index("c")
        r0 = (wid // 4) * TH
        c0 = (wid % 4) * (2 * TW)
        zeros16 = jnp.zeros((16,), jnp.float32)
        ones16 = jnp.ones((16,), jnp.float32)
        cp_in = pltpu.async_copy(
            idx_hbm.at[pl.ds(r0, TH), pl.ds(c0, 2 * TW)], idx_v, sem
        )
        for i in range(2 * BINS // 16):
            hist_v[pl.ds(i * 16, 16)] = zeros16
        cp_in.wait()

        @pl.loop(0, TH, step=8)
        def _(r):
            for r8 in range(8):
                for cc in range(2 * TW // 16):
                    v = idx_v[r + r8, pl.ds(cc * 16, 16)]
                    b = jnp.bitwise_and(v, 2 * BINS - 1)
                    plsc.addupdate_scatter(hist_v, [b], ones16)

        for k in range(2):
            base = k * BINS
            excess = jnp.float32(0.0)
            for i in range(BINS // 16):
                h = hist_v[pl.ds(base + i * 16, 16)]
                excess = excess + jnp.sum(jnp.maximum(h - CLIP_LIMIT, 0.0))
            redist = excess * (1.0 / BINS)
            off = jnp.float32(0.0)
            for i in range(BINS // 16):
                h = hist_v[pl.ds(base + i * 16, 16)]
                h2 = jnp.minimum(h, CLIP_LIMIT) + redist
                c = plsc.cumsum(h2) + off
                cdf_v[pl.ds(base + i * 16, 16)] = c
                off = jnp.max(c)
            total = off
            for i in range(BINS // 16):
                cdf_v[pl.ds(base + i * 16, 16)] = (
                    cdf_v[pl.ds(base + i * 16, 16)] / total
                )

        @pl.loop(0, TH, step=8)
        def _(r):
            for r8 in range(8):
                for cc in range(2 * TW // 16):
                    v = idx_v[r + r8, pl.ds(cc * 16, 16)]
                    i2 = jnp.right_shift(v, 9)
                    eq_v[r + r8, pl.ds(cc * 16, 16)] = plsc.load_gather(
                        cdf_v, [i2]
                    )

        pltpu.async_copy(
            eq_v, eq_hbm.at[pl.ds(r0, TH), pl.ds(c0, 2 * TW)], sem
        ).wait()

    return sck(idx_packed)


@jax.jit
def kernel(img):
    idx = pl.pallas_call(
        _k_prep,
        grid=(GRID,),
        in_specs=[pl.BlockSpec((3, TH, W), lambda i: (0, i, 0))],
        out_specs=pl.BlockSpec((TH, W), lambda i: (i, 0)),
        out_shape=jax.ShapeDtypeStruct((H, W), jnp.int32),
    )(img)
    eq = _sc_equalize(idx)
    out = pl.pallas_call(
        _k_final,
        grid=(GRID,),
        in_specs=[
            pl.BlockSpec((3, TH, W), lambda i: (0, i, 0)),
            pl.BlockSpec((TH, W), lambda i: (i, 0)),
        ],
        out_specs=pl.BlockSpec((3, TH, W), lambda i: (0, i, 0)),
        out_shape=jax.ShapeDtypeStruct((3, H, W), jnp.float32),
    )(img, eq)
    return out


# speedup vs baseline: 1.3517x; 1.1142x over previous
import dataclasses
import functools

import jax
import jax.numpy as jnp
from jax import lax
from jax.experimental import pallas as pl
from jax.experimental.pallas import tpu as pltpu
from jax.experimental.pallas import tpu_sc as plsc

H = W = 512
TH = TW = 64
GRID = 8
BINS = 256
CLIP_LIMIT = 4.0
CONTRAST_FACTOR = 1.15
BLEND_FACTOR = 0.65


def _row_tile_reduce(x, red, fill):
    r0 = red(x, axis=0, keepdims=True)
    lane = lax.broadcasted_iota(jnp.int32, r0.shape, 1)
    g = lane // TW
    out = jnp.zeros_like(r0)
    for t in range(W // TW):
        m = g == t
        out = jnp.where(m, red(jnp.where(m, r0, fill)), out)
    return out


def _per_tile_reduce(x, red, fill):
    n = x.shape[0] // TH
    if n == 1:
        return _row_tile_reduce(x, red, fill)
    rows = [
        _row_tile_reduce(x[i * TH:(i + 1) * TH], red, fill) for i in range(n)
    ]
    row = lax.broadcasted_iota(jnp.int32, (x.shape[0], 1), 0)
    out = rows[0]
    for i in range(1, n):
        out = jnp.where(row < i * TH, out, rows[i])
    return out


def _k_prep(img_ref, idx_ref):
    img = img_ref[...]
    y = 0.299 * img[0] + 0.587 * img[1] + 0.114 * img[2]
    i1 = jnp.clip(jnp.floor(y * BINS).astype(jnp.int32), 0, BINS - 1)
    tmin = _per_tile_reduce(y, jnp.min, jnp.float32(jnp.inf))
    tmax = _per_tile_reduce(y, jnp.max, jnp.float32(-jnp.inf))
    denom = jnp.where(tmax > tmin, tmax - tmin, 1.0)
    normalized = (y - tmin) / denom
    i2 = jnp.clip((normalized * 255).astype(jnp.int32), 0, 255)
    lane = lax.broadcasted_iota(jnp.int32, y.shape, y.ndim - 1)
    toff = (lane // TW) % 2 * BINS
    idx_ref[...] = (i1 + toff) | ((i2 + toff) << 9)


def _k_final(img_ref, eq_ref, out_ref):
    img = img_ref[...]
    y = 0.299 * img[0] + 0.587 * img[1] + 0.114 * img[2]
    eq = eq_ref[...]
    tmin = _per_tile_reduce(y, jnp.min, jnp.float32(jnp.inf))
    tmax = _per_tile_reduce(y, jnp.max, jnp.float32(-jnp.inf))
    has_range = tmax > tmin
    mean_val = _per_tile_reduce(eq, jnp.sum, jnp.float32(0.0)) * (
        1.0 / (TH * TW)
    )
    eq2 = (eq - mean_val) * CONTRAST_FACTOR + mean_val
    eq2 = jnp.clip(eq2, 0.0, 1.0)
    blended = y + BLEND_FACTOR * (eq2 - y)
    y_new = jnp.where(has_range, blended, y)
    safe = jnp.where(y > 0.01, y, 1.0)
    ratio = jnp.where(y > 0.01, y_new / safe, 1.0)
    out_ref[...] = jnp.clip(img * ratio[None, :, :], 0.0, 1.0)


def _sc_equalize(idx_packed):
    mesh = plsc.VectorSubcoreMesh(core_axis_name="c", subcore_axis_name="s")
    cp = pltpu.CompilerParams()
    if "needs_layout_passes" in pltpu.CompilerParams.__dataclass_fields__:
        cp = dataclasses.replace(cp, needs_layout_passes=False)

    @functools.partial(
        pl.kernel,
        out_type=jax.ShapeDtypeStruct((H, W), jnp.float32),
        mesh=mesh,
        compiler_params=cp,
        scratch_types=[
            pltpu.VMEM((TH, 2 * TW), jnp.int32),
            pltpu.VMEM((TH, 2 * TW), jnp.float32),
            pltpu.VMEM((2 * BINS,), jnp.float32),
            pltpu.VMEM((2 * BINS,), jnp.float32),
            pltpu.SemaphoreType.DMA,
        ],
    )
    def sck(idx_hbm, eq_hbm, idx_v, eq_v, hist_v, cdf_v, sem):
        wid = lax.axis_index("s") * 2 + lax.axis_index("c")
        r0 = (wid // 4) * TH
        c0 = (wid % 4) * (2 * TW)
        zeros16 = jnp.zeros((16,), jnp.float32)
        ones16 = jnp.ones((16,), jnp.float32)
        cp_in = pltpu.async_copy(
            idx_hbm.at[pl.ds(r0, TH), pl.ds(c0, 2 * TW)], idx_v, sem
        )
        for i in range(2 * BINS // 16):
            hist_v[pl.ds(i * 16, 16)] = zeros16
        cp_in.wait()

        @pl.loop(0, TH, step=8)
        def _(r):
            for r8 in range(8):
                for cc in range(2 * TW // 16):
                    v = idx_v[r + r8, pl.ds(cc * 16, 16)]
                    b = jnp.bitwise_and(v, 2 * BINS - 1)
                    plsc.addupdate_scatter(hist_v, [b], ones16)

        for k in range(2):
            base = k * BINS
            excess = jnp.float32(0.0)
            for i in range(BINS // 16):
                h = hist_v[pl.ds(base + i * 16, 16)]
                excess = excess + jnp.sum(jnp.maximum(h - CLIP_LIMIT, 0.0))
            redist = excess * (1.0 / BINS)
            off = jnp.float32(0.0)
            for i in range(BINS // 16):
                h = hist_v[pl.ds(base + i * 16, 16)]
                h2 = jnp.minimum(h, CLIP_LIMIT) + redist
                c = plsc.cumsum(h2) + off
                cdf_v[pl.ds(base + i * 16, 16)] = c
                off = jnp.max(c)
            total = off
            for i in range(BINS // 16):
                cdf_v[pl.ds(base + i * 16, 16)] = (
                    cdf_v[pl.ds(base + i * 16, 16)] / total
                )

        @pl.loop(0, TH, step=8)
        def _(r):
            for r8 in range(8):
                for cc in range(2 * TW // 16):
                    v = idx_v[r + r8, pl.ds(cc * 16, 16)]
                    i2 = jnp.right_shift(v, 9)
                    eq_v[r + r8, pl.ds(cc * 16, 16)] = plsc.load_gather(
                        cdf_v, [i2]
                    )

        pltpu.async_copy(
            eq_v, eq_hbm.at[pl.ds(r0, TH), pl.ds(c0, 2 * TW)], sem
        ).wait()

    return sck(idx_packed)


@jax.jit
def kernel(img):
    idx = pl.pallas_call(
        _k_prep,
        grid=(GRID // 2,),
        in_specs=[pl.BlockSpec((3, 2 * TH, W), lambda i: (0, i, 0))],
        out_specs=pl.BlockSpec((2 * TH, W), lambda i: (i, 0)),
        out_shape=jax.ShapeDtypeStruct((H, W), jnp.int32),
    )(img)
    eq = _sc_equalize(idx)
    out = pl.pallas_call(
        _k_final,
        grid=(GRID // 2,),
        in_specs=[
            pl.BlockSpec((3, 2 * TH, W), lambda i: (0, i, 0)),
            pl.BlockSpec((2 * TH, W), lambda i: (i, 0)),
        ],
        out_specs=pl.BlockSpec((3, 2 * TH, W), lambda i: (0, i, 0)),
        out_shape=jax.ShapeDtypeStruct((3, H, W), jnp.float32),
    )(img, eq)
    return out


# speedup vs baseline: 1.3972x; 1.0337x over previous
import dataclasses
import functools

import jax
import jax.numpy as jnp
from jax import lax
from jax.experimental import pallas as pl
from jax.experimental.pallas import tpu as pltpu
from jax.experimental.pallas import tpu_sc as plsc

H = W = 512
TH = TW = 64
GRID = 8
BINS = 256
CLIP_LIMIT = 4.0
CONTRAST_FACTOR = 1.15
BLEND_FACTOR = 0.65


def _row_tile_reduce(x, red, fill):
    r0 = red(x, axis=0, keepdims=True)
    lane = lax.broadcasted_iota(jnp.int32, r0.shape, 1)
    g = lane // TW
    out = jnp.zeros_like(r0)
    for t in range(W // TW):
        m = g == t
        out = jnp.where(m, red(jnp.where(m, r0, fill)), out)
    return out


def _per_tile_reduce(x, red, fill):
    n = x.shape[0] // TH
    if n == 1:
        return _row_tile_reduce(x, red, fill)
    rows = [
        _row_tile_reduce(x[i * TH:(i + 1) * TH], red, fill) for i in range(n)
    ]
    row = lax.broadcasted_iota(jnp.int32, (x.shape[0], 1), 0)
    out = rows[0]
    for i in range(1, n):
        out = jnp.where(row < i * TH, out, rows[i])
    return out


def _k_prep(img_ref, idx_ref):
    img = img_ref[...]
    y = 0.299 * img[0] + 0.587 * img[1] + 0.114 * img[2]
    i1 = jnp.clip(jnp.floor(y * BINS).astype(jnp.int32), 0, BINS - 1)
    tmin = _per_tile_reduce(y, jnp.min, jnp.float32(jnp.inf))
    tmax = _per_tile_reduce(y, jnp.max, jnp.float32(-jnp.inf))
    denom = jnp.where(tmax > tmin, tmax - tmin, 1.0)
    normalized = (y - tmin) / denom
    i2 = jnp.clip((normalized * 255).astype(jnp.int32), 0, 255)
    lane = lax.broadcasted_iota(jnp.int32, y.shape, y.ndim - 1)
    toff = (lane // TW) % 2 * BINS
    idx_ref[...] = (i1 + toff) | ((i2 + toff) << 9)


def _k_final(img_ref, eq_ref, out_ref):
    img = img_ref[...]
    y = 0.299 * img[0] + 0.587 * img[1] + 0.114 * img[2]
    eq = eq_ref[...]
    tmin = _per_tile_reduce(y, jnp.min, jnp.float32(jnp.inf))
    tmax = _per_tile_reduce(y, jnp.max, jnp.float32(-jnp.inf))
    has_range = tmax > tmin
    mean_val = _per_tile_reduce(eq, jnp.sum, jnp.float32(0.0)) * (
        1.0 / (TH * TW)
    )
    eq2 = (eq - mean_val) * CONTRAST_FACTOR + mean_val
    eq2 = jnp.clip(eq2, 0.0, 1.0)
    blended = y + BLEND_FACTOR * (eq2 - y)
    y_new = jnp.where(has_range, blended, y)
    safe = jnp.where(y > 0.01, y, 1.0)
    ratio = jnp.where(y > 0.01, y_new / safe, 1.0)
    out_ref[...] = jnp.clip(img * ratio[None, :, :], 0.0, 1.0)


def _sc_equalize(idx_packed):
    mesh = plsc.VectorSubcoreMesh(core_axis_name="c", subcore_axis_name="s")
    cp = pltpu.CompilerParams()
    if "needs_layout_passes" in pltpu.CompilerParams.__dataclass_fields__:
        cp = dataclasses.replace(cp, needs_layout_passes=False)

    @functools.partial(
        pl.kernel,
        out_type=jax.ShapeDtypeStruct((H, W), jnp.float32),
        mesh=mesh,
        compiler_params=cp,
        scratch_types=[
            pltpu.VMEM((TH, 2 * TW), jnp.int32),
            pltpu.VMEM((TH, 2 * TW), jnp.float32),
            pltpu.VMEM((2 * BINS,), jnp.float32),
            pltpu.VMEM((2 * BINS,), jnp.float32),
            pltpu.SemaphoreType.DMA,
        ],
    )
    def sck(idx_hbm, eq_hbm, idx_v, eq_v, hist_v, cdf_v, sem):
        wid = lax.axis_index("s") * 2 + lax.axis_index("c")
        r0 = (wid // 4) * TH
        c0 = (wid % 4) * (2 * TW)
        zeros16 = jnp.zeros((16,), jnp.float32)
        ones16 = jnp.ones((16,), jnp.float32)
        cp_in = pltpu.async_copy(
            idx_hbm.at[pl.ds(r0, TH), pl.ds(c0, 2 * TW)], idx_v, sem
        )
        for i in range(2 * BINS // 16):
            hist_v[pl.ds(i * 16, 16)] = zeros16
        cp_in.wait()

        @pl.loop(0, TH, step=8)
        def _(r):
            for r8 in range(8):
                for cc in range(2 * TW // 16):
                    v = idx_v[r + r8, pl.ds(cc * 16, 16)]
                    b = jnp.bitwise_and(v, 2 * BINS - 1)
                    plsc.addupdate_scatter(hist_v, [b], ones16)

        for k in range(2):
            base = k * BINS
            excess = jnp.float32(0.0)
            for i in range(BINS // 16):
                h = hist_v[pl.ds(base + i * 16, 16)]
                excess = excess + jnp.sum(jnp.maximum(h - CLIP_LIMIT, 0.0))
            redist = excess * (1.0 / BINS)
            off = jnp.float32(0.0)
            for i in range(BINS // 16):
                h = hist_v[pl.ds(base + i * 16, 16)]
                h2 = jnp.minimum(h, CLIP_LIMIT) + redist
                c = plsc.cumsum(h2) + off
                cdf_v[pl.ds(base + i * 16, 16)] = c
                off = jnp.max(c)
            total = off
            for i in range(BINS // 16):
                cdf_v[pl.ds(base + i * 16, 16)] = (
                    cdf_v[pl.ds(base + i * 16, 16)] / total
                )

        @pl.loop(0, TH, step=8)
        def _(r):
            for r8 in range(8):
                for cc in range(2 * TW // 16):
                    v = idx_v[r + r8, pl.ds(cc * 16, 16)]
                    i2 = jnp.right_shift(v, 9)
                    eq_v[r + r8, pl.ds(cc * 16, 16)] = plsc.load_gather(
                        cdf_v, [i2]
                    )

        pltpu.async_copy(
            eq_v, eq_hbm.at[pl.ds(r0, TH), pl.ds(c0, 2 * TW)], sem
        ).wait()

    return sck(idx_packed)


@jax.jit
def kernel(img):
    idx = pl.pallas_call(
        _k_prep,
        grid=(GRID // 4,),
        in_specs=[pl.BlockSpec((3, 4 * TH, W), lambda i: (0, i, 0))],
        out_specs=pl.BlockSpec((4 * TH, W), lambda i: (i, 0)),
        out_shape=jax.ShapeDtypeStruct((H, W), jnp.int32),
    )(img)
    eq = _sc_equalize(idx)
    out = pl.pallas_call(
        _k_final,
        grid=(GRID // 4,),
        in_specs=[
            pl.BlockSpec((3, 4 * TH, W), lambda i: (0, i, 0)),
            pl.BlockSpec((4 * TH, W), lambda i: (i, 0)),
        ],
        out_specs=pl.BlockSpec((3, 4 * TH, W), lambda i: (0, i, 0)),
        out_shape=jax.ShapeDtypeStruct((3, H, W), jnp.float32),
    )(img, eq)
    return out
